# baseline (speedup 1.0000x reference)
import jax
import jax.numpy as jnp
from jax import lax
from jax.experimental import pallas as pl
from jax.experimental.pallas import tpu as pltpu
from jax.experimental.pallas import tpu_sc as plsc

BATCH = 4096
HIST = 200
LPAD = 208
C0, C1 = 112, 96
EMBED = 64
LANES = 16
NWORKERS = 32
ROWS_PER_W = BATCH // NWORKERS
NDC = EMBED // LANES

_BCAST_DNUMS = lax.GatherDimensionNumbers(
    offset_dims=(), collapsed_slice_dims=(0,), start_index_map=(0,))


def _bcast_lane(v, j):
    return lax.gather(v, jnp.full((LANES, 1), j, jnp.int32), _BCAST_DNUMS,
                      slice_sizes=(1,),
                      mode=lax.GatherScatterMode.PROMISE_IN_BOUNDS)


def _gelu(v):
    u = 0.7978845608028654 * (v + 0.044715 * v * v * v)
    e = jnp.exp(2.0 * u)
    t = 1.0 - 2.0 / (e + 1.0)
    return 0.5 * v * (1.0 + t)


def _sc_body(x_hbm, idx_hbm, tbl_hbm, out_hbm,
             x_v, idx_v, rows0, rows1, out_v, sem0, sem1):
    wid = lax.axis_index("s") * 2 + lax.axis_index("c")
    inbase = pl.multiple_of(wid * (ROWS_PER_W * LPAD), 128)
    obase = pl.multiple_of(wid * (ROWS_PER_W * EMBED), 128)
    pltpu.sync_copy(x_hbm.at[pl.ds(inbase, ROWS_PER_W * LPAD)], x_v)
    pltpu.sync_copy(idx_hbm.at[pl.ds(inbase, ROWS_PER_W * LPAD)], idx_v)

    def gather(b, off, n, dst, sem):
        start = pl.multiple_of(b * LPAD + off, 16)
        return pltpu.make_async_copy(tbl_hbm.at[idx_v.at[pl.ds(start, n)]],
                                     dst, sem)

    gather(0, 0, C0, rows0, sem0).start()

    def accum_half(b, off, n, rows, acc):
        for k in range(n // LANES):
            t0 = off + k * LANES
            s = pl.multiple_of(b * LPAD + t0, 16)
            w = x_v[pl.ds(s, LANES)]
            iv = idx_v[pl.ds(s, LANES)]
            w = jnp.where(iv != 0, w, 0.0)
            for j in range(LANES):
                wb = _bcast_lane(w, j)
                r = k * LANES + j
                for dc in range(NDC):
                    acc[dc] = acc[dc] + wb * rows[r, pl.ds(dc * LANES, LANES)]
        return acc

    def body(b, carry):
        gather(b, C0, C1, rows1, sem1).start()
        gather(b, 0, C0, rows0, sem0).wait()
        acc = [jnp.zeros((LANES,), jnp.float32) for _ in range(NDC)]
        acc = accum_half(b, 0, C0, rows0, acc)
        bn = jnp.minimum(b + 1, ROWS_PER_W - 1)
        gather(bn, 0, C0, rows0, sem0).start()
        gather(b, C0, C1, rows1, sem1).wait()
        acc = accum_half(b, C0, C1, rows1, acc)
        for dc in range(NDC):
            o = pl.multiple_of(b * EMBED + dc * LANES, 16)
            out_v[pl.ds(o, LANES)] = _gelu(acc[dc])
        return carry

    lax.fori_loop(0, ROWS_PER_W, body, 0)
    gather(ROWS_PER_W - 1, 0, C0, rows0, sem0).wait()
    pltpu.sync_copy(out_v, out_hbm.at[pl.ds(obase, ROWS_PER_W * EMBED)])


def kernel(x, x_ind, table):
    xp = jnp.pad(x, ((0, 0), (0, LPAD - HIST))).reshape(-1)
    ip = jnp.pad(x_ind.astype(jnp.int32), ((0, 0), (0, LPAD - HIST))).reshape(-1)
    run = pl.kernel(
        _sc_body,
        out_type=jax.ShapeDtypeStruct((BATCH * EMBED,), jnp.float32),
        scratch_types=[
            pltpu.VMEM((ROWS_PER_W * LPAD,), jnp.float32),
            pltpu.VMEM((ROWS_PER_W * LPAD,), jnp.int32),
            pltpu.VMEM((C0, EMBED), jnp.float32),
            pltpu.VMEM((C1, EMBED), jnp.float32),
            pltpu.VMEM((ROWS_PER_W * EMBED,), jnp.float32),
            pltpu.SemaphoreType.DMA,
            pltpu.SemaphoreType.DMA,
        ],
        mesh=plsc.VectorSubcoreMesh(core_axis_name="c", subcore_axis_name="s"),
        compiler_params=pltpu.CompilerParams(use_tc_tiling_on_sc=False),
    )
    return run(xp, ip, table).reshape(BATCH, EMBED)
